# baseline (speedup 1.0000x reference)
import functools

import jax
import jax.numpy as jnp
from jax import lax
from jax.experimental import pallas as pl
from jax.experimental.pallas import tpu as pltpu
from jax.experimental.pallas import tpu_sc as plsc

NC = 2
NS = 16
LANES = 16
NW = NC * NS

DIM = 32
NNEG = 20
CB = 64

WB = 1024
GP = 245
Q = GP * WB


def _pack_body(x0, x1, x2, x3, o_ref):
    eye = jnp.eye(DIM, dtype=jnp.float32)

    def t(x):
        return lax.dot_general(
            x[...], eye, (((0,), (0,)), ((), ())),
            preferred_element_type=jnp.float32,
            precision=lax.Precision.HIGHEST)

    o_ref[...] = jnp.concatenate([t(x0), t(x1), t(x2), t(x3)], axis=1)


def _pack_table(X):
    Xt = jnp.transpose(X)
    nblk = X.shape[0] // WB
    packed = pl.pallas_call(
        _pack_body,
        grid=(GP,),
        in_specs=[pl.BlockSpec(
            (DIM, WB), lambda g, a=a: (0, jnp.minimum(g + a * GP, nblk)))
                  for a in range(4)],
        out_specs=pl.BlockSpec((WB, 128), lambda g: (g, 0)),
        out_shape=jax.ShapeDtypeStruct((Q, 128), jnp.float32),
        compiler_params=pltpu.CompilerParams(
            fuse_transposed_lhs_in_matmul=True),
    )(Xt, Xt, Xt, Xt)
    return jnp.reshape(packed, (4 * Q, DIM))


def _sc_scores_builder(B):
    assert B % NW == 0
    bpw = B // NW
    assert bpw % CB == 0
    nch = bpw // CB
    ngather = (CB * NNEG) // 128

    mesh = plsc.VectorSubcoreMesh(core_axis_name="c", subcore_axis_name="s",
                                  num_cores=NC, num_subcores=NS)

    @functools.partial(
        pl.kernel,
        out_type=(jax.ShapeDtypeStruct((B,), jnp.float32),
                  jax.ShapeDtypeStruct((B,), jnp.float32)),
        mesh=mesh,
        compiler_params=pltpu.CompilerParams(needs_layout_passes=False,
                                             use_tc_tiling_on_sc=False),
        scratch_types=[
            pltpu.VMEM((bpw,), jnp.int32),
            pltpu.VMEM((bpw,), jnp.int32),
            pltpu.VMEM((bpw * NNEG,), jnp.int32),
            pltpu.VMEM((CB, DIM), jnp.float32),
            pltpu.VMEM((CB, DIM), jnp.float32),
            pltpu.VMEM((CB * NNEG, DIM), jnp.float32),
            pltpu.VMEM((bpw,), jnp.float32),
            pltpu.VMEM((bpw,), jnp.float32),
            pltpu.SemaphoreType.DMA,
        ],
    )
    def sc_scores(U_hbm, V_hbm, uidx_hbm, pidx_hbm, nidx_hbm,
                  pos_out, neg_out,
                  uidx_v, pidx_v, nidx_v, u_rows, p_rows, n_rows,
                  pos_sv, neg_sv, sem):
        wid = lax.axis_index("s") * NC + lax.axis_index("c")
        base = wid * bpw

        pltpu.sync_copy(uidx_hbm.at[pl.ds(base, bpw)], uidx_v)
        pltpu.sync_copy(pidx_hbm.at[pl.ds(base, bpw)], pidx_v)
        pltpu.sync_copy(nidx_hbm.at[pl.ds(base * NNEG, bpw * NNEG)], nidx_v)

        def remap(ref, count):
            def body(j, _):
                v = ref[pl.ds(j * LANES, LANES)]
                a = ((v >= Q).astype(jnp.int32)
                     + (v >= 2 * Q).astype(jnp.int32)
                     + (v >= 3 * Q).astype(jnp.int32))
                ref[pl.ds(j * LANES, LANES)] = (v - a * Q) * 4 + a
                return 0
            lax.fori_loop(0, count // LANES, body, 0)

        remap(uidx_v, bpw)
        remap(pidx_v, bpw)
        remap(nidx_v, bpw * NNEG)

        lane = lax.iota(jnp.int32, LANES)

        def chunk_body(c, _):
            cb = c * CB
            copies = [
                pltpu.async_copy(U_hbm.at[uidx_v.at[pl.ds(cb, CB)]], u_rows, sem),
                pltpu.async_copy(V_hbm.at[pidx_v.at[pl.ds(cb, CB)]], p_rows, sem),
            ]
            for j in range(ngather):
                copies.append(pltpu.async_copy(
                    V_hbm.at[nidx_v.at[pl.ds(cb * NNEG + j * 128, 128)]],
                    n_rows.at[pl.ds(j * 128, 128)], sem))
            for cp in copies:
                cp.wait()

            for g in range(CB // LANES):
                b_ids = lane + (g * LANES)
                r_base = b_ids * NNEG

                def d_body(d, carry):
                    pos_vec, neg_vec = carry
                    d_ids = jnp.full((LANES,), d, jnp.int32)
                    u_d = plsc.load_gather(u_rows, [b_ids, d_ids])
                    p_d = plsc.load_gather(p_rows, [b_ids, d_ids])
                    nacc = plsc.load_gather(n_rows, [r_base, d_ids])
                    for n in range(1, NNEG):
                        nacc = nacc + plsc.load_gather(
                            n_rows, [r_base + n, d_ids])
                    return (pos_vec + u_d * p_d, neg_vec + u_d * nacc)

                zero = jnp.zeros((LANES,), jnp.float32)
                pos_vec, neg_vec = lax.fori_loop(0, DIM, d_body, (zero, zero))
                pos_sv[pl.ds(cb + g * LANES, LANES)] = pos_vec
                neg_sv[pl.ds(cb + g * LANES, LANES)] = neg_vec
            return 0

        lax.fori_loop(0, nch, chunk_body, 0)

        pltpu.sync_copy(pos_sv, pos_out.at[pl.ds(base, bpw)])
        pltpu.sync_copy(neg_sv, neg_out.at[pl.ds(base, bpw)])

    return sc_scores


def _tc_loss_body(pos_ref, neg_ref, bs_ref, out_ref):
    pos = pos_ref[...]
    neg = -neg_ref[...]
    lp = jnp.minimum(pos, 0.0) - jnp.log1p(jnp.exp(-jnp.abs(pos)))
    ln = jnp.minimum(neg, 0.0) - jnp.log1p(jnp.exp(-jnp.abs(neg)))
    out_ref[0, 0] = -jnp.sum(lp + ln) / bs_ref[0]


def kernel(U, V, u_pos, v_pos, v_neg, batch_size):
    B = u_pos.shape[0]
    uidx = u_pos.astype(jnp.int32)
    pidx = v_pos.astype(jnp.int32)
    nidx = v_neg.astype(jnp.int32).reshape(-1)

    Up = _pack_table(U)
    Vp = _pack_table(V)
    pos_s, neg_s = _sc_scores_builder(B)(Up, Vp, uidx, pidx, nidx)

    rows = B // 128
    bs = jnp.asarray(batch_size, jnp.float32).reshape(1)
    loss = pl.pallas_call(
        _tc_loss_body,
        out_shape=jax.ShapeDtypeStruct((1, 1), jnp.float32),
        in_specs=[
            pl.BlockSpec(memory_space=pltpu.VMEM),
            pl.BlockSpec(memory_space=pltpu.VMEM),
            pl.BlockSpec(memory_space=pltpu.SMEM),
        ],
        out_specs=pl.BlockSpec(memory_space=pltpu.SMEM),
    )(pos_s.reshape(rows, 128), neg_s.reshape(rows, 128), bs)
    return loss.reshape(())


# speedup vs baseline: 2.2316x; 2.2316x over previous
import functools

import jax
import jax.numpy as jnp
from jax import lax
from jax.experimental import pallas as pl
from jax.experimental.pallas import tpu as pltpu
from jax.experimental.pallas import tpu_sc as plsc

NC = 2
NS = 16
LANES = 16
NW = NC * NS

DIM = 32
NNEG = 20
CB = 64

WB = 1024
GP = 245
Q = GP * WB


def _pack_body(x0, x1, x2, x3, o_ref):
    stacked = jnp.concatenate(
        [x0[...], x1[...], x2[...], x3[...]], axis=0)
    o_ref[...] = jnp.transpose(stacked)


def _pack_table(X):
    Xt = jnp.transpose(X)
    nblk = X.shape[0] // WB
    packed = pl.pallas_call(
        _pack_body,
        grid=(GP,),
        in_specs=[pl.BlockSpec(
            (DIM, WB), lambda g, a=a: (0, jnp.minimum(g + a * GP, nblk)))
                  for a in range(4)],
        out_specs=pl.BlockSpec((WB, 128), lambda g: (g, 0)),
        out_shape=jax.ShapeDtypeStruct((Q, 128), jnp.float32),
    )(Xt, Xt, Xt, Xt)
    return jnp.reshape(packed, (4 * Q, DIM))


def _sc_scores_builder(B):
    assert B % NW == 0
    bpw = B // NW
    assert bpw % CB == 0
    nch = bpw // CB
    ngather = (CB * NNEG) // 128

    mesh = plsc.VectorSubcoreMesh(core_axis_name="c", subcore_axis_name="s",
                                  num_cores=NC, num_subcores=NS)

    @functools.partial(
        pl.kernel,
        out_type=(jax.ShapeDtypeStruct((B,), jnp.float32),
                  jax.ShapeDtypeStruct((B,), jnp.float32)),
        mesh=mesh,
        compiler_params=pltpu.CompilerParams(needs_layout_passes=False,
                                             use_tc_tiling_on_sc=False),
        scratch_types=[
            pltpu.VMEM((bpw,), jnp.int32),
            pltpu.VMEM((bpw,), jnp.int32),
            pltpu.VMEM((bpw * NNEG,), jnp.int32),
            pltpu.VMEM((CB, DIM), jnp.float32),
            pltpu.VMEM((CB, DIM), jnp.float32),
            pltpu.VMEM((CB * NNEG, DIM), jnp.float32),
            pltpu.VMEM((bpw,), jnp.float32),
            pltpu.VMEM((bpw,), jnp.float32),
            pltpu.SemaphoreType.DMA,
        ],
    )
    def sc_scores(U_hbm, V_hbm, uidx_hbm, pidx_hbm, nidx_hbm,
                  pos_out, neg_out,
                  uidx_v, pidx_v, nidx_v, u_rows, p_rows, n_rows,
                  pos_sv, neg_sv, sem):
        wid = lax.axis_index("s") * NC + lax.axis_index("c")
        base = wid * bpw

        pltpu.sync_copy(uidx_hbm.at[pl.ds(base, bpw)], uidx_v)
        pltpu.sync_copy(pidx_hbm.at[pl.ds(base, bpw)], pidx_v)
        pltpu.sync_copy(nidx_hbm.at[pl.ds(base * NNEG, bpw * NNEG)], nidx_v)

        def remap(ref, count):
            def body(j, _):
                v = ref[pl.ds(j * LANES, LANES)]
                a = ((v >= Q).astype(jnp.int32)
                     + (v >= 2 * Q).astype(jnp.int32)
                     + (v >= 3 * Q).astype(jnp.int32))
                ref[pl.ds(j * LANES, LANES)] = (v - a * Q) * 4 + a
                return 0
            lax.fori_loop(0, count // LANES, body, 0)

        remap(uidx_v, bpw)
        remap(pidx_v, bpw)
        remap(nidx_v, bpw * NNEG)

        lane = lax.iota(jnp.int32, LANES)

        def chunk_body(c, _):
            cb = c * CB
            copies = [
                pltpu.async_copy(U_hbm.at[uidx_v.at[pl.ds(cb, CB)]], u_rows, sem),
                pltpu.async_copy(V_hbm.at[pidx_v.at[pl.ds(cb, CB)]], p_rows, sem),
            ]
            for j in range(ngather):
                copies.append(pltpu.async_copy(
                    V_hbm.at[nidx_v.at[pl.ds(cb * NNEG + j * 128, 128)]],
                    n_rows.at[pl.ds(j * 128, 128)], sem))
            for cp in copies:
                cp.wait()

            for g in range(CB // LANES):
                b_ids = lane + (g * LANES)
                r_base = b_ids * NNEG

                def d_body(d, carry):
                    pos_vec, neg_vec = carry
                    d_ids = jnp.full((LANES,), d, jnp.int32)
                    u_d = plsc.load_gather(u_rows, [b_ids, d_ids])
                    p_d = plsc.load_gather(p_rows, [b_ids, d_ids])
                    nacc = plsc.load_gather(n_rows, [r_base, d_ids])
                    for n in range(1, NNEG):
                        nacc = nacc + plsc.load_gather(
                            n_rows, [r_base + n, d_ids])
                    return (pos_vec + u_d * p_d, neg_vec + u_d * nacc)

                zero = jnp.zeros((LANES,), jnp.float32)
                pos_vec, neg_vec = lax.fori_loop(0, DIM, d_body, (zero, zero))
                pos_sv[pl.ds(cb + g * LANES, LANES)] = pos_vec
                neg_sv[pl.ds(cb + g * LANES, LANES)] = neg_vec
            return 0

        lax.fori_loop(0, nch, chunk_body, 0)

        pltpu.sync_copy(pos_sv, pos_out.at[pl.ds(base, bpw)])
        pltpu.sync_copy(neg_sv, neg_out.at[pl.ds(base, bpw)])

    return sc_scores


def _tc_loss_body(pos_ref, neg_ref, bs_ref, out_ref):
    pos = pos_ref[...]
    neg = -neg_ref[...]
    lp = jnp.minimum(pos, 0.0) - jnp.log1p(jnp.exp(-jnp.abs(pos)))
    ln = jnp.minimum(neg, 0.0) - jnp.log1p(jnp.exp(-jnp.abs(neg)))
    out_ref[0, 0] = -jnp.sum(lp + ln) / bs_ref[0]


def kernel(U, V, u_pos, v_pos, v_neg, batch_size):
    B = u_pos.shape[0]
    uidx = u_pos.astype(jnp.int32)
    pidx = v_pos.astype(jnp.int32)
    nidx = v_neg.astype(jnp.int32).reshape(-1)

    Up = _pack_table(U)
    Vp = _pack_table(V)
    pos_s, neg_s = _sc_scores_builder(B)(Up, Vp, uidx, pidx, nidx)

    rows = B // 128
    bs = jnp.asarray(batch_size, jnp.float32).reshape(1)
    loss = pl.pallas_call(
        _tc_loss_body,
        out_shape=jax.ShapeDtypeStruct((1, 1), jnp.float32),
        in_specs=[
            pl.BlockSpec(memory_space=pltpu.VMEM),
            pl.BlockSpec(memory_space=pltpu.VMEM),
            pl.BlockSpec(memory_space=pltpu.SMEM),
        ],
        out_specs=pl.BlockSpec(memory_space=pltpu.SMEM),
    )(pos_s.reshape(rows, 128), neg_s.reshape(rows, 128), bs)
    return loss.reshape(())


# speedup vs baseline: 2.8550x; 1.2794x over previous
import functools

import jax
import jax.numpy as jnp
from jax import lax
from jax.experimental import pallas as pl
from jax.experimental.pallas import tpu as pltpu
from jax.experimental.pallas import tpu_sc as plsc

NC = 2
NS = 16
LANES = 16
NW = NC * NS

DIM = 32
NNEG = 20
CB = 128

WB = 1024
GP = 245
Q = GP * WB


def _pack_body(x0, x1, x2, x3, o_ref):
    stacked = jnp.concatenate(
        [x0[...], x1[...], x2[...], x3[...]], axis=0)
    o_ref[...] = jnp.transpose(stacked)


def _pack_table(X):
    Xt = jnp.transpose(X)
    nblk = X.shape[0] // WB
    packed = pl.pallas_call(
        _pack_body,
        grid=(GP,),
        in_specs=[pl.BlockSpec(
            (DIM, WB), lambda g, a=a: (0, jnp.minimum(g + a * GP, nblk)))
                  for a in range(4)],
        out_specs=pl.BlockSpec((WB, 128), lambda g: (g, 0)),
        out_shape=jax.ShapeDtypeStruct((Q, 128), jnp.float32),
    )(Xt, Xt, Xt, Xt)
    return jnp.reshape(packed, (4 * Q, DIM))


def _sc_scores_builder(B):
    assert B % NW == 0
    bpw = B // NW
    assert bpw % CB == 0
    nch = bpw // CB

    mesh = plsc.VectorSubcoreMesh(core_axis_name="c", subcore_axis_name="s",
                                  num_cores=NC, num_subcores=NS)

    @functools.partial(
        pl.kernel,
        out_type=(jax.ShapeDtypeStruct((B,), jnp.float32),
                  jax.ShapeDtypeStruct((B,), jnp.float32)),
        mesh=mesh,
        compiler_params=pltpu.CompilerParams(needs_layout_passes=False,
                                             use_tc_tiling_on_sc=False),
        scratch_types=[
            pltpu.VMEM((bpw,), jnp.int32),
            pltpu.VMEM((bpw,), jnp.int32),
            pltpu.VMEM((NNEG, bpw), jnp.int32),
            pltpu.VMEM((CB, DIM), jnp.float32),
            pltpu.VMEM((CB, DIM), jnp.float32),
            pltpu.VMEM((CB, DIM), jnp.float32),
            pltpu.VMEM((bpw,), jnp.float32),
            pltpu.VMEM((bpw,), jnp.float32),
            pltpu.SemaphoreType.DMA,
        ],
    )
    def sc_scores(U_hbm, V_hbm, uidx_hbm, pidx_hbm, nidxT_hbm,
                  pos_out, neg_out,
                  uidx_v, pidx_v, nidxT_v, u_rows, p_rows, n_sum,
                  pos_sv, neg_sv, sem):
        wid = lax.axis_index("s") * NC + lax.axis_index("c")
        base = wid * bpw

        pltpu.sync_copy(uidx_hbm.at[pl.ds(base, bpw)], uidx_v)
        pltpu.sync_copy(pidx_hbm.at[pl.ds(base, bpw)], pidx_v)
        for n in range(NNEG):
            pltpu.sync_copy(nidxT_hbm.at[pl.ds(n * B + base, bpw)],
                            nidxT_v.at[n])

        def remap16(v):
            a = ((v >= Q).astype(jnp.int32)
                 + (v >= 2 * Q).astype(jnp.int32)
                 + (v >= 3 * Q).astype(jnp.int32))
            return (v - a * Q) * 4 + a

        def remap1d(ref):
            def body(j, _):
                ref[pl.ds(j * LANES, LANES)] = remap16(
                    ref[pl.ds(j * LANES, LANES)])
                return 0
            lax.fori_loop(0, bpw // LANES, body, 0)

        remap1d(uidx_v)
        remap1d(pidx_v)
        for n in range(NNEG):
            def bodyn(j, _, n=n):
                nidxT_v[n, pl.ds(j * LANES, LANES)] = remap16(
                    nidxT_v[n, pl.ds(j * LANES, LANES)])
                return 0
            lax.fori_loop(0, bpw // LANES, bodyn, 0)

        lane = lax.iota(jnp.int32, LANES)
        zero = jnp.zeros((LANES,), jnp.float32)

        def chunk_body(c, _):
            cb = c * CB

            def zbody(i, _):
                n_sum[i, pl.ds(0, LANES)] = zero
                n_sum[i, pl.ds(LANES, LANES)] = zero
                return 0
            lax.fori_loop(0, CB, zbody, 0)

            copies = [
                pltpu.async_copy(U_hbm.at[uidx_v.at[pl.ds(cb, CB)]], u_rows, sem),
                pltpu.async_copy(V_hbm.at[pidx_v.at[pl.ds(cb, CB)]], p_rows, sem),
            ]
            for n in range(NNEG):
                copies.append(pltpu.async_copy(
                    V_hbm.at[nidxT_v.at[n, pl.ds(cb, CB)]],
                    n_sum, sem, add=True))
            for cp in copies:
                cp.wait()

            for g in range(CB // LANES):
                b_ids = lane + (g * LANES)

                def d_body(d, carry):
                    pos_vec, neg_vec = carry
                    d_ids = jnp.full((LANES,), d, jnp.int32)
                    u_d = plsc.load_gather(u_rows, [b_ids, d_ids])
                    p_d = plsc.load_gather(p_rows, [b_ids, d_ids])
                    ns_d = plsc.load_gather(n_sum, [b_ids, d_ids])
                    return (pos_vec + u_d * p_d, neg_vec + u_d * ns_d)

                pos_vec, neg_vec = lax.fori_loop(0, DIM, d_body, (zero, zero))
                pos_sv[pl.ds(cb + g * LANES, LANES)] = pos_vec
                neg_sv[pl.ds(cb + g * LANES, LANES)] = neg_vec
            return 0

        lax.fori_loop(0, nch, chunk_body, 0)

        pltpu.sync_copy(pos_sv, pos_out.at[pl.ds(base, bpw)])
        pltpu.sync_copy(neg_sv, neg_out.at[pl.ds(base, bpw)])

    return sc_scores


def _tc_loss_body(pos_ref, neg_ref, bs_ref, out_ref):
    pos = pos_ref[...]
    neg = -neg_ref[...]
    lp = jnp.minimum(pos, 0.0) - jnp.log1p(jnp.exp(-jnp.abs(pos)))
    ln = jnp.minimum(neg, 0.0) - jnp.log1p(jnp.exp(-jnp.abs(neg)))
    out_ref[0, 0] = -jnp.sum(lp + ln) / bs_ref[0]


def kernel(U, V, u_pos, v_pos, v_neg, batch_size):
    B = u_pos.shape[0]
    uidx = u_pos.astype(jnp.int32)
    pidx = v_pos.astype(jnp.int32)
    nidxT = jnp.transpose(v_neg.astype(jnp.int32)).reshape(-1)

    Up = _pack_table(U)
    Vp = _pack_table(V)
    pos_s, neg_s = _sc_scores_builder(B)(Up, Vp, uidx, pidx, nidxT)

    rows = B // 128
    bs = jnp.asarray(batch_size, jnp.float32).reshape(1)
    loss = pl.pallas_call(
        _tc_loss_body,
        out_shape=jax.ShapeDtypeStruct((1, 1), jnp.float32),
        in_specs=[
            pl.BlockSpec(memory_space=pltpu.VMEM),
            pl.BlockSpec(memory_space=pltpu.VMEM),
            pl.BlockSpec(memory_space=pltpu.SMEM),
        ],
        out_specs=pl.BlockSpec(memory_space=pltpu.SMEM),
    )(pos_s.reshape(rows, 128), neg_s.reshape(rows, 128), bs)
    return loss.reshape(())


# speedup vs baseline: 3.7802x; 1.3241x over previous
import functools

import jax
import jax.numpy as jnp
from jax import lax
from jax.experimental import pallas as pl
from jax.experimental.pallas import tpu as pltpu
from jax.experimental.pallas import tpu_sc as plsc

NC = 2
NS = 16
LANES = 16
NW = NC * NS

DIM = 32
NNEG = 20
CB = 128

WB = 2048
GP = 123
Q = GP * WB


def _pack_body(x0, x1, x2, x3, o_ref):
    stacked = jnp.concatenate(
        [x0[...], x1[...], x2[...], x3[...]], axis=0)
    o_ref[...] = jnp.transpose(stacked)


def _pack_table(X):
    Xt = jnp.transpose(X)
    nblk = X.shape[0] // WB
    packed = pl.pallas_call(
        _pack_body,
        grid=(GP,),
        in_specs=[pl.BlockSpec(
            (DIM, WB), lambda g, a=a: (0, jnp.minimum(g + a * GP, nblk)))
                  for a in range(4)],
        out_specs=pl.BlockSpec((WB, 128), lambda g: (g, 0)),
        out_shape=jax.ShapeDtypeStruct((Q, 128), jnp.float32),
    )(Xt, Xt, Xt, Xt)
    return jnp.reshape(packed, (4 * Q, DIM))


def _sc_scores_builder(B):
    assert B % NW == 0
    bpw = B // NW
    assert bpw % CB == 0
    nch = bpw // CB

    mesh = plsc.VectorSubcoreMesh(core_axis_name="c", subcore_axis_name="s",
                                  num_cores=NC, num_subcores=NS)

    @functools.partial(
        pl.kernel,
        out_type=(jax.ShapeDtypeStruct((B,), jnp.float32),
                  jax.ShapeDtypeStruct((B,), jnp.float32)),
        mesh=mesh,
        compiler_params=pltpu.CompilerParams(needs_layout_passes=False,
                                             use_tc_tiling_on_sc=False),
        scratch_types=[
            pltpu.VMEM((bpw,), jnp.int32),
            pltpu.VMEM((bpw,), jnp.int32),
            pltpu.VMEM((NNEG, bpw), jnp.int32),
            pltpu.VMEM((CB, DIM), jnp.float32),
            pltpu.VMEM((CB, DIM), jnp.float32),
            pltpu.VMEM((CB, DIM), jnp.float32),
            pltpu.VMEM((bpw,), jnp.float32),
            pltpu.VMEM((bpw,), jnp.float32),
            pltpu.SemaphoreType.DMA,
        ],
    )
    def sc_scores(U_hbm, V_hbm, uidx_hbm, pidx_hbm, nidxT_hbm,
                  pos_out, neg_out,
                  uidx_v, pidx_v, nidxT_v, u_rows, p_rows, n_sum,
                  pos_sv, neg_sv, sem):
        wid = lax.axis_index("s") * NC + lax.axis_index("c")
        base = wid * bpw

        pltpu.sync_copy(uidx_hbm.at[pl.ds(base, bpw)], uidx_v)
        pltpu.sync_copy(pidx_hbm.at[pl.ds(base, bpw)], pidx_v)
        for n in range(NNEG):
            pltpu.sync_copy(nidxT_hbm.at[pl.ds(n * B + base, bpw)],
                            nidxT_v.at[n])

        def remap16(v):
            a = ((v >= Q).astype(jnp.int32)
                 + (v >= 2 * Q).astype(jnp.int32)
                 + (v >= 3 * Q).astype(jnp.int32))
            return (v - a * Q) * 4 + a

        def remap1d(ref):
            def body(j, _):
                ref[pl.ds(j * LANES, LANES)] = remap16(
                    ref[pl.ds(j * LANES, LANES)])
                return 0
            lax.fori_loop(0, bpw // LANES, body, 0)

        remap1d(uidx_v)
        remap1d(pidx_v)
        for n in range(NNEG):
            def bodyn(j, _, n=n):
                nidxT_v[n, pl.ds(j * LANES, LANES)] = remap16(
                    nidxT_v[n, pl.ds(j * LANES, LANES)])
                return 0
            lax.fori_loop(0, bpw // LANES, bodyn, 0)

        lane = lax.iota(jnp.int32, LANES)
        zero = jnp.zeros((LANES,), jnp.float32)

        def chunk_body(c, _):
            cb = c * CB

            def zbody(i, _):
                n_sum[i, pl.ds(0, LANES)] = zero
                n_sum[i, pl.ds(LANES, LANES)] = zero
                return 0
            lax.fori_loop(0, CB, zbody, 0)

            copies = [
                pltpu.async_copy(U_hbm.at[uidx_v.at[pl.ds(cb, CB)]], u_rows, sem),
                pltpu.async_copy(V_hbm.at[pidx_v.at[pl.ds(cb, CB)]], p_rows, sem),
            ]
            for n in range(NNEG):
                copies.append(pltpu.async_copy(
                    V_hbm.at[nidxT_v.at[n, pl.ds(cb, CB)]],
                    n_sum, sem, add=True))
            for cp in copies:
                cp.wait()

            for g in range(CB // LANES):
                b_ids = lane + (g * LANES)

                def d_body(d, carry):
                    pos_vec, neg_vec = carry
                    d_ids = jnp.full((LANES,), d, jnp.int32)
                    u_d = plsc.load_gather(u_rows, [b_ids, d_ids])
                    p_d = plsc.load_gather(p_rows, [b_ids, d_ids])
                    ns_d = plsc.load_gather(n_sum, [b_ids, d_ids])
                    return (pos_vec + u_d * p_d, neg_vec + u_d * ns_d)

                pos_vec, neg_vec = lax.fori_loop(0, DIM, d_body, (zero, zero))
                pos_sv[pl.ds(cb + g * LANES, LANES)] = pos_vec
                neg_sv[pl.ds(cb + g * LANES, LANES)] = neg_vec
            return 0

        lax.fori_loop(0, nch, chunk_body, 0)

        pltpu.sync_copy(pos_sv, pos_out.at[pl.ds(base, bpw)])
        pltpu.sync_copy(neg_sv, neg_out.at[pl.ds(base, bpw)])

    return sc_scores


def _tc_loss_body(pos_ref, neg_ref, bs_ref, out_ref):
    pos = pos_ref[...]
    neg = -neg_ref[...]
    lp = jnp.minimum(pos, 0.0) - jnp.log1p(jnp.exp(-jnp.abs(pos)))
    ln = jnp.minimum(neg, 0.0) - jnp.log1p(jnp.exp(-jnp.abs(neg)))
    out_ref[0, 0] = -jnp.sum(lp + ln) / bs_ref[0]


def kernel(U, V, u_pos, v_pos, v_neg, batch_size):
    B = u_pos.shape[0]
    uidx = u_pos.astype(jnp.int32)
    pidx = v_pos.astype(jnp.int32)
    nidxT = jnp.transpose(v_neg.astype(jnp.int32)).reshape(-1)

    Up = _pack_table(U)
    Vp = _pack_table(V)
    pos_s, neg_s = _sc_scores_builder(B)(Up, Vp, uidx, pidx, nidxT)

    rows = B // 128
    bs = jnp.asarray(batch_size, jnp.float32).reshape(1)
    loss = pl.pallas_call(
        _tc_loss_body,
        out_shape=jax.ShapeDtypeStruct((1, 1), jnp.float32),
        in_specs=[
            pl.BlockSpec(memory_space=pltpu.VMEM),
            pl.BlockSpec(memory_space=pltpu.VMEM),
            pl.BlockSpec(memory_space=pltpu.SMEM),
        ],
        out_specs=pl.BlockSpec(memory_space=pltpu.SMEM),
    )(pos_s.reshape(rows, 128), neg_s.reshape(rows, 128), bs)
    return loss.reshape(())


# speedup vs baseline: 4.7489x; 1.2563x over previous
import functools

import jax
import jax.numpy as jnp
from jax import lax
from jax.experimental import pallas as pl
from jax.experimental.pallas import tpu as pltpu
from jax.experimental.pallas import tpu_sc as plsc

NC = 2
NS = 16
LANES = 16
NW = NC * NS

DIM = 32
NNEG = 20
CB = 128

WB = 4096
GP = 62
Q = GP * WB


def _pack_body(x0, x1, x2, x3, o_ref):
    stacked = jnp.concatenate(
        [x0[...], x1[...], x2[...], x3[...]], axis=0)
    o_ref[...] = jnp.transpose(stacked)


def _pack_table(X):
    Xt = jnp.transpose(X)
    nblk = X.shape[0] // WB
    packed = pl.pallas_call(
        _pack_body,
        grid=(GP,),
        in_specs=[pl.BlockSpec(
            (DIM, WB), lambda g, a=a: (0, jnp.minimum(g + a * GP, nblk)))
                  for a in range(4)],
        out_specs=pl.BlockSpec((WB, 128), lambda g: (g, 0)),
        out_shape=jax.ShapeDtypeStruct((Q, 128), jnp.float32),
    )(Xt, Xt, Xt, Xt)
    return jnp.reshape(packed, (4 * Q, DIM))


def _sc_scores_builder(B):
    assert B % NW == 0
    bpw = B // NW
    assert bpw % CB == 0
    nch = bpw // CB

    mesh = plsc.VectorSubcoreMesh(core_axis_name="c", subcore_axis_name="s",
                                  num_cores=NC, num_subcores=NS)

    @functools.partial(
        pl.kernel,
        out_type=(jax.ShapeDtypeStruct((B,), jnp.float32),
                  jax.ShapeDtypeStruct((B,), jnp.float32)),
        mesh=mesh,
        compiler_params=pltpu.CompilerParams(needs_layout_passes=False,
                                             use_tc_tiling_on_sc=False),
        scratch_types=[
            pltpu.VMEM((bpw,), jnp.int32),
            pltpu.VMEM((bpw,), jnp.int32),
            pltpu.VMEM((NNEG, bpw), jnp.int32),
            pltpu.VMEM((CB, DIM), jnp.float32),
            pltpu.VMEM((CB, DIM), jnp.float32),
            pltpu.VMEM((CB, DIM), jnp.float32),
            pltpu.VMEM((bpw,), jnp.float32),
            pltpu.VMEM((bpw,), jnp.float32),
            pltpu.SemaphoreType.DMA,
        ],
    )
    def sc_scores(U_hbm, V_hbm, uidx_hbm, pidx_hbm, nidxT_hbm,
                  pos_out, neg_out,
                  uidx_v, pidx_v, nidxT_v, u_rows, p_rows, n_sum,
                  pos_sv, neg_sv, sem):
        wid = lax.axis_index("s") * NC + lax.axis_index("c")
        base = wid * bpw

        pltpu.sync_copy(uidx_hbm.at[pl.ds(base, bpw)], uidx_v)
        pltpu.sync_copy(pidx_hbm.at[pl.ds(base, bpw)], pidx_v)
        for n in range(NNEG):
            pltpu.sync_copy(nidxT_hbm.at[pl.ds(n * B + base, bpw)],
                            nidxT_v.at[n])

        def remap16(v):
            a = ((v >= Q).astype(jnp.int32)
                 + (v >= 2 * Q).astype(jnp.int32)
                 + (v >= 3 * Q).astype(jnp.int32))
            return (v - a * Q) * 4 + a

        def remap1d(ref):
            def body(j, _):
                ref[pl.ds(j * LANES, LANES)] = remap16(
                    ref[pl.ds(j * LANES, LANES)])
                return 0
            lax.fori_loop(0, bpw // LANES, body, 0)

        remap1d(uidx_v)
        remap1d(pidx_v)
        for n in range(NNEG):
            def bodyn(j, _, n=n):
                nidxT_v[n, pl.ds(j * LANES, LANES)] = remap16(
                    nidxT_v[n, pl.ds(j * LANES, LANES)])
                return 0
            lax.fori_loop(0, bpw // LANES, bodyn, 0)

        lane = lax.iota(jnp.int32, LANES)
        zero = jnp.zeros((LANES,), jnp.float32)

        def chunk_body(c, _):
            cb = c * CB

            def zbody(i, _):
                n_sum[i, pl.ds(0, LANES)] = zero
                n_sum[i, pl.ds(LANES, LANES)] = zero
                return 0
            lax.fori_loop(0, CB, zbody, 0)

            copies = [
                pltpu.async_copy(U_hbm.at[uidx_v.at[pl.ds(cb, CB)]], u_rows, sem),
                pltpu.async_copy(V_hbm.at[pidx_v.at[pl.ds(cb, CB)]], p_rows, sem),
            ]
            for n in range(NNEG):
                copies.append(pltpu.async_copy(
                    V_hbm.at[nidxT_v.at[n, pl.ds(cb, CB)]],
                    n_sum, sem, add=True))
            for cp in copies:
                cp.wait()

            for g in range(CB // LANES):
                b_ids = lane + (g * LANES)

                def d_body(d, carry):
                    pos_vec, neg_vec = carry
                    d_ids = jnp.full((LANES,), d, jnp.int32)
                    u_d = plsc.load_gather(u_rows, [b_ids, d_ids])
                    p_d = plsc.load_gather(p_rows, [b_ids, d_ids])
                    ns_d = plsc.load_gather(n_sum, [b_ids, d_ids])
                    return (pos_vec + u_d * p_d, neg_vec + u_d * ns_d)

                pos_vec, neg_vec = lax.fori_loop(0, DIM, d_body, (zero, zero))
                pos_sv[pl.ds(cb + g * LANES, LANES)] = pos_vec
                neg_sv[pl.ds(cb + g * LANES, LANES)] = neg_vec
            return 0

        lax.fori_loop(0, nch, chunk_body, 0)

        pltpu.sync_copy(pos_sv, pos_out.at[pl.ds(base, bpw)])
        pltpu.sync_copy(neg_sv, neg_out.at[pl.ds(base, bpw)])

    return sc_scores


def _tc_loss_body(pos_ref, neg_ref, bs_ref, out_ref):
    pos = pos_ref[...]
    neg = -neg_ref[...]
    lp = jnp.minimum(pos, 0.0) - jnp.log1p(jnp.exp(-jnp.abs(pos)))
    ln = jnp.minimum(neg, 0.0) - jnp.log1p(jnp.exp(-jnp.abs(neg)))
    out_ref[0, 0] = -jnp.sum(lp + ln) / bs_ref[0]


def kernel(U, V, u_pos, v_pos, v_neg, batch_size):
    B = u_pos.shape[0]
    uidx = u_pos.astype(jnp.int32)
    pidx = v_pos.astype(jnp.int32)
    nidxT = jnp.transpose(v_neg.astype(jnp.int32)).reshape(-1)

    Up = _pack_table(U)
    Vp = _pack_table(V)
    pos_s, neg_s = _sc_scores_builder(B)(Up, Vp, uidx, pidx, nidxT)

    rows = B // 128
    bs = jnp.asarray(batch_size, jnp.float32).reshape(1)
    loss = pl.pallas_call(
        _tc_loss_body,
        out_shape=jax.ShapeDtypeStruct((1, 1), jnp.float32),
        in_specs=[
            pl.BlockSpec(memory_space=pltpu.VMEM),
            pl.BlockSpec(memory_space=pltpu.VMEM),
            pl.BlockSpec(memory_space=pltpu.SMEM),
        ],
        out_specs=pl.BlockSpec(memory_space=pltpu.SMEM),
    )(pos_s.reshape(rows, 128), neg_s.reshape(rows, 128), bs)
    return loss.reshape(())


# speedup vs baseline: 5.2983x; 1.1157x over previous
import functools

import jax
import jax.numpy as jnp
from jax import lax
from jax.experimental import pallas as pl
from jax.experimental.pallas import tpu as pltpu
from jax.experimental.pallas import tpu_sc as plsc

NC = 2
NS = 16
LANES = 16
NW = NC * NS

DIM = 32
NNEG = 20
CB = 128

WB = 8192
GP = 31
Q = GP * WB


def _pack_body(x0, x1, x2, x3, o_ref):
    stacked = jnp.concatenate(
        [x0[...], x1[...], x2[...], x3[...]], axis=0)
    o_ref[...] = jnp.transpose(stacked)


def _pack_table(X):
    Xt = jnp.transpose(X)
    nblk = X.shape[0] // WB
    packed = pl.pallas_call(
        _pack_body,
        grid=(GP,),
        in_specs=[pl.BlockSpec(
            (DIM, WB), lambda g, a=a: (0, jnp.minimum(g + a * GP, nblk)))
                  for a in range(4)],
        out_specs=pl.BlockSpec((WB, 128), lambda g: (g, 0)),
        out_shape=jax.ShapeDtypeStruct((Q, 128), jnp.float32),
    )(Xt, Xt, Xt, Xt)
    return jnp.reshape(packed, (4 * Q, DIM))


def _sc_scores_builder(B):
    assert B % NW == 0
    bpw = B // NW
    assert bpw % CB == 0
    nch = bpw // CB

    mesh = plsc.VectorSubcoreMesh(core_axis_name="c", subcore_axis_name="s",
                                  num_cores=NC, num_subcores=NS)

    @functools.partial(
        pl.kernel,
        out_type=(jax.ShapeDtypeStruct((B,), jnp.float32),
                  jax.ShapeDtypeStruct((B,), jnp.float32)),
        mesh=mesh,
        compiler_params=pltpu.CompilerParams(needs_layout_passes=False,
                                             use_tc_tiling_on_sc=False),
        scratch_types=[
            pltpu.VMEM((bpw,), jnp.int32),
            pltpu.VMEM((bpw,), jnp.int32),
            pltpu.VMEM((NNEG, bpw), jnp.int32),
            pltpu.VMEM((CB, DIM), jnp.float32),
            pltpu.VMEM((CB, DIM), jnp.float32),
            pltpu.VMEM((CB, DIM), jnp.float32),
            pltpu.VMEM((bpw,), jnp.float32),
            pltpu.VMEM((bpw,), jnp.float32),
            pltpu.SemaphoreType.DMA,
        ],
    )
    def sc_scores(U_hbm, V_hbm, uidx_hbm, pidx_hbm, nidxT_hbm,
                  pos_out, neg_out,
                  uidx_v, pidx_v, nidxT_v, u_rows, p_rows, n_sum,
                  pos_sv, neg_sv, sem):
        wid = lax.axis_index("s") * NC + lax.axis_index("c")
        base = wid * bpw

        pltpu.sync_copy(uidx_hbm.at[pl.ds(base, bpw)], uidx_v)
        pltpu.sync_copy(pidx_hbm.at[pl.ds(base, bpw)], pidx_v)
        for n in range(NNEG):
            pltpu.sync_copy(nidxT_hbm.at[pl.ds(n * B + base, bpw)],
                            nidxT_v.at[n])

        def remap16(v):
            a = ((v >= Q).astype(jnp.int32)
                 + (v >= 2 * Q).astype(jnp.int32)
                 + (v >= 3 * Q).astype(jnp.int32))
            return (v - a * Q) * 4 + a

        def remap1d(ref):
            def body(j, _):
                ref[pl.ds(j * LANES, LANES)] = remap16(
                    ref[pl.ds(j * LANES, LANES)])
                return 0
            lax.fori_loop(0, bpw // LANES, body, 0)

        remap1d(uidx_v)
        remap1d(pidx_v)
        for n in range(NNEG):
            def bodyn(j, _, n=n):
                nidxT_v[n, pl.ds(j * LANES, LANES)] = remap16(
                    nidxT_v[n, pl.ds(j * LANES, LANES)])
                return 0
            lax.fori_loop(0, bpw // LANES, bodyn, 0)

        lane = lax.iota(jnp.int32, LANES)
        zero = jnp.zeros((LANES,), jnp.float32)

        def chunk_body(c, _):
            cb = c * CB

            def zbody(i, _):
                n_sum[i, pl.ds(0, LANES)] = zero
                n_sum[i, pl.ds(LANES, LANES)] = zero
                return 0
            lax.fori_loop(0, CB, zbody, 0)

            copies = [
                pltpu.async_copy(U_hbm.at[uidx_v.at[pl.ds(cb, CB)]], u_rows, sem),
                pltpu.async_copy(V_hbm.at[pidx_v.at[pl.ds(cb, CB)]], p_rows, sem),
            ]
            for n in range(NNEG):
                copies.append(pltpu.async_copy(
                    V_hbm.at[nidxT_v.at[n, pl.ds(cb, CB)]],
                    n_sum, sem, add=True))
            for cp in copies:
                cp.wait()

            for g in range(CB // LANES):
                b_ids = lane + (g * LANES)

                def d_body(d, carry):
                    pos_vec, neg_vec = carry
                    d_ids = jnp.full((LANES,), d, jnp.int32)
                    u_d = plsc.load_gather(u_rows, [b_ids, d_ids])
                    p_d = plsc.load_gather(p_rows, [b_ids, d_ids])
                    ns_d = plsc.load_gather(n_sum, [b_ids, d_ids])
                    return (pos_vec + u_d * p_d, neg_vec + u_d * ns_d)

                pos_vec, neg_vec = lax.fori_loop(0, DIM, d_body, (zero, zero))
                pos_sv[pl.ds(cb + g * LANES, LANES)] = pos_vec
                neg_sv[pl.ds(cb + g * LANES, LANES)] = neg_vec
            return 0

        lax.fori_loop(0, nch, chunk_body, 0)

        pltpu.sync_copy(pos_sv, pos_out.at[pl.ds(base, bpw)])
        pltpu.sync_copy(neg_sv, neg_out.at[pl.ds(base, bpw)])

    return sc_scores


def _tc_loss_body(pos_ref, neg_ref, bs_ref, out_ref):
    pos = pos_ref[...]
    neg = -neg_ref[...]
    lp = jnp.minimum(pos, 0.0) - jnp.log1p(jnp.exp(-jnp.abs(pos)))
    ln = jnp.minimum(neg, 0.0) - jnp.log1p(jnp.exp(-jnp.abs(neg)))
    out_ref[0, 0] = -jnp.sum(lp + ln) / bs_ref[0]


def kernel(U, V, u_pos, v_pos, v_neg, batch_size):
    B = u_pos.shape[0]
    uidx = u_pos.astype(jnp.int32)
    pidx = v_pos.astype(jnp.int32)
    nidxT = jnp.transpose(v_neg.astype(jnp.int32)).reshape(-1)

    Up = _pack_table(U)
    Vp = _pack_table(V)
    pos_s, neg_s = _sc_scores_builder(B)(Up, Vp, uidx, pidx, nidxT)

    rows = B // 128
    bs = jnp.asarray(batch_size, jnp.float32).reshape(1)
    loss = pl.pallas_call(
        _tc_loss_body,
        out_shape=jax.ShapeDtypeStruct((1, 1), jnp.float32),
        in_specs=[
            pl.BlockSpec(memory_space=pltpu.VMEM),
            pl.BlockSpec(memory_space=pltpu.VMEM),
            pl.BlockSpec(memory_space=pltpu.SMEM),
        ],
        out_specs=pl.BlockSpec(memory_space=pltpu.SMEM),
    )(pos_s.reshape(rows, 128), neg_s.reshape(rows, 128), bs)
    return loss.reshape(())


# speedup vs baseline: 5.3853x; 1.0164x over previous
import functools

import jax
import jax.numpy as jnp
from jax import lax
from jax.experimental import pallas as pl
from jax.experimental.pallas import tpu as pltpu
from jax.experimental.pallas import tpu_sc as plsc

NC = 2
NS = 16
LANES = 16
NW = NC * NS

DIM = 32
NNEG = 20
CB = 128

WB = 16384
GP = 16
Q = GP * WB


def _pack_body(x0, x1, x2, x3, o_ref):
    stacked = jnp.concatenate(
        [x0[...], x1[...], x2[...], x3[...]], axis=0)
    o_ref[...] = jnp.transpose(stacked)


def _pack_table(X):
    Xt = jnp.transpose(X)
    nblk = X.shape[0] // WB
    packed = pl.pallas_call(
        _pack_body,
        grid=(GP,),
        in_specs=[pl.BlockSpec(
            (DIM, WB), lambda g, a=a: (0, jnp.minimum(g + a * GP, nblk)))
                  for a in range(4)],
        out_specs=pl.BlockSpec((WB, 128), lambda g: (g, 0)),
        out_shape=jax.ShapeDtypeStruct((Q, 128), jnp.float32),
    )(Xt, Xt, Xt, Xt)
    return jnp.reshape(packed, (4 * Q, DIM))


def _sc_scores_builder(B):
    assert B % NW == 0
    bpw = B // NW
    assert bpw % CB == 0
    nch = bpw // CB

    mesh = plsc.VectorSubcoreMesh(core_axis_name="c", subcore_axis_name="s",
                                  num_cores=NC, num_subcores=NS)

    @functools.partial(
        pl.kernel,
        out_type=(jax.ShapeDtypeStruct((B,), jnp.float32),
                  jax.ShapeDtypeStruct((B,), jnp.float32)),
        mesh=mesh,
        compiler_params=pltpu.CompilerParams(needs_layout_passes=False,
                                             use_tc_tiling_on_sc=False),
        scratch_types=[
            pltpu.VMEM((bpw,), jnp.int32),
            pltpu.VMEM((bpw,), jnp.int32),
            pltpu.VMEM((NNEG, bpw), jnp.int32),
            pltpu.VMEM((CB, DIM), jnp.float32),
            pltpu.VMEM((CB, DIM), jnp.float32),
            pltpu.VMEM((CB, DIM), jnp.float32),
            pltpu.VMEM((bpw,), jnp.float32),
            pltpu.VMEM((bpw,), jnp.float32),
            pltpu.SemaphoreType.DMA,
        ],
    )
    def sc_scores(U_hbm, V_hbm, uidx_hbm, pidx_hbm, nidxT_hbm,
                  pos_out, neg_out,
                  uidx_v, pidx_v, nidxT_v, u_rows, p_rows, n_sum,
                  pos_sv, neg_sv, sem):
        wid = lax.axis_index("s") * NC + lax.axis_index("c")
        base = wid * bpw

        pltpu.sync_copy(uidx_hbm.at[pl.ds(base, bpw)], uidx_v)
        pltpu.sync_copy(pidx_hbm.at[pl.ds(base, bpw)], pidx_v)
        for n in range(NNEG):
            pltpu.sync_copy(nidxT_hbm.at[pl.ds(n * B + base, bpw)],
                            nidxT_v.at[n])

        def remap16(v):
            a = ((v >= Q).astype(jnp.int32)
                 + (v >= 2 * Q).astype(jnp.int32)
                 + (v >= 3 * Q).astype(jnp.int32))
            return (v - a * Q) * 4 + a

        def remap1d(ref):
            def body(j, _):
                ref[pl.ds(j * LANES, LANES)] = remap16(
                    ref[pl.ds(j * LANES, LANES)])
                return 0
            lax.fori_loop(0, bpw // LANES, body, 0)

        remap1d(uidx_v)
        remap1d(pidx_v)
        for n in range(NNEG):
            def bodyn(j, _, n=n):
                nidxT_v[n, pl.ds(j * LANES, LANES)] = remap16(
                    nidxT_v[n, pl.ds(j * LANES, LANES)])
                return 0
            lax.fori_loop(0, bpw // LANES, bodyn, 0)

        lane = lax.iota(jnp.int32, LANES)
        zero = jnp.zeros((LANES,), jnp.float32)

        def chunk_body(c, _):
            cb = c * CB

            def zbody(i, _):
                n_sum[i, pl.ds(0, LANES)] = zero
                n_sum[i, pl.ds(LANES, LANES)] = zero
                return 0
            lax.fori_loop(0, CB, zbody, 0)

            copies = [
                pltpu.async_copy(U_hbm.at[uidx_v.at[pl.ds(cb, CB)]], u_rows, sem),
                pltpu.async_copy(V_hbm.at[pidx_v.at[pl.ds(cb, CB)]], p_rows, sem),
            ]
            for n in range(NNEG):
                copies.append(pltpu.async_copy(
                    V_hbm.at[nidxT_v.at[n, pl.ds(cb, CB)]],
                    n_sum, sem, add=True))
            for cp in copies:
                cp.wait()

            for g in range(CB // LANES):
                b_ids = lane + (g * LANES)

                def d_body(d, carry):
                    pos_vec, neg_vec = carry
                    d_ids = jnp.full((LANES,), d, jnp.int32)
                    u_d = plsc.load_gather(u_rows, [b_ids, d_ids])
                    p_d = plsc.load_gather(p_rows, [b_ids, d_ids])
                    ns_d = plsc.load_gather(n_sum, [b_ids, d_ids])
                    return (pos_vec + u_d * p_d, neg_vec + u_d * ns_d)

                pos_vec, neg_vec = lax.fori_loop(0, DIM, d_body, (zero, zero))
                pos_sv[pl.ds(cb + g * LANES, LANES)] = pos_vec
                neg_sv[pl.ds(cb + g * LANES, LANES)] = neg_vec
            return 0

        lax.fori_loop(0, nch, chunk_body, 0)

        pltpu.sync_copy(pos_sv, pos_out.at[pl.ds(base, bpw)])
        pltpu.sync_copy(neg_sv, neg_out.at[pl.ds(base, bpw)])

    return sc_scores


def _tc_loss_body(pos_ref, neg_ref, bs_ref, out_ref):
    pos = pos_ref[...]
    neg = -neg_ref[...]
    lp = jnp.minimum(pos, 0.0) - jnp.log1p(jnp.exp(-jnp.abs(pos)))
    ln = jnp.minimum(neg, 0.0) - jnp.log1p(jnp.exp(-jnp.abs(neg)))
    out_ref[0, 0] = -jnp.sum(lp + ln) / bs_ref[0]


def kernel(U, V, u_pos, v_pos, v_neg, batch_size):
    B = u_pos.shape[0]
    uidx = u_pos.astype(jnp.int32)
    pidx = v_pos.astype(jnp.int32)
    nidxT = jnp.transpose(v_neg.astype(jnp.int32)).reshape(-1)

    Up = _pack_table(U)
    Vp = _pack_table(V)
    pos_s, neg_s = _sc_scores_builder(B)(Up, Vp, uidx, pidx, nidxT)

    rows = B // 128
    bs = jnp.asarray(batch_size, jnp.float32).reshape(1)
    loss = pl.pallas_call(
        _tc_loss_body,
        out_shape=jax.ShapeDtypeStruct((1, 1), jnp.float32),
        in_specs=[
            pl.BlockSpec(memory_space=pltpu.VMEM),
            pl.BlockSpec(memory_space=pltpu.VMEM),
            pl.BlockSpec(memory_space=pltpu.SMEM),
        ],
        out_specs=pl.BlockSpec(memory_space=pltpu.SMEM),
    )(pos_s.reshape(rows, 128), neg_s.reshape(rows, 128), bs)
    return loss.reshape(())


# speedup vs baseline: 5.7132x; 1.0609x over previous
import functools

import jax
import jax.numpy as jnp
from jax import lax
from jax.experimental import pallas as pl
from jax.experimental.pallas import tpu as pltpu
from jax.experimental.pallas import tpu_sc as plsc

NC = 2
NS = 16
LANES = 16
NW = NC * NS

DIM = 32
NNEG = 20
CB = 128

WB = 16384
GP = 16
Q = GP * WB


def _pack_body(x0, x1, x2, x3, o_ref):
    stacked = jnp.concatenate(
        [x0[...], x1[...], x2[...], x3[...]], axis=0)
    o_ref[...] = jnp.transpose(stacked)


def _pack_table(X):
    Xt = jnp.transpose(X)
    nblk = X.shape[0] // WB
    packed = pl.pallas_call(
        _pack_body,
        grid=(GP,),
        in_specs=[pl.BlockSpec(
            (DIM, WB), lambda g, a=a: (0, jnp.minimum(g + a * GP, nblk)))
                  for a in range(4)],
        out_specs=pl.BlockSpec((WB, 128), lambda g: (g, 0)),
        out_shape=jax.ShapeDtypeStruct((Q, 128), jnp.float32),
    )(Xt, Xt, Xt, Xt)
    return jnp.reshape(packed, (4 * Q, DIM))


def _sc_mesh():
    return plsc.VectorSubcoreMesh(core_axis_name="c", subcore_axis_name="s",
                                  num_cores=NC, num_subcores=NS)


def _remap16(v):
    a = ((v >= Q).astype(jnp.int32)
         + (v >= 2 * Q).astype(jnp.int32)
         + (v >= 3 * Q).astype(jnp.int32))
    return (v - a * Q) * 4 + a


_SC_PARAMS = pltpu.CompilerParams(needs_layout_passes=False,
                                  use_tc_tiling_on_sc=False)


def _sc_vrows_builder(B):
    bpw = B // NW
    nch = bpw // CB

    @functools.partial(
        pl.kernel,
        out_type=(jax.ShapeDtypeStruct((B, DIM), jnp.float32),
                  jax.ShapeDtypeStruct((B, DIM), jnp.float32)),
        mesh=_sc_mesh(),
        compiler_params=_SC_PARAMS,
        scratch_types=[
            pltpu.VMEM((bpw,), jnp.int32),
            pltpu.VMEM((NNEG, bpw), jnp.int32),
            pltpu.VMEM((CB, DIM), jnp.float32),
            pltpu.VMEM((CB, DIM), jnp.float32),
            pltpu.SemaphoreType.DMA,
        ],
    )
    def sc_vrows(V_hbm, pidx_hbm, nidxT_hbm, prow_out, nsum_out,
                 pidx_v, nidxT_v, p_rows, n_sum, sem):
        wid = lax.axis_index("s") * NC + lax.axis_index("c")
        base = wid * bpw

        pltpu.sync_copy(pidx_hbm.at[pl.ds(base, bpw)], pidx_v)
        for n in range(NNEG):
            pltpu.sync_copy(nidxT_hbm.at[pl.ds(n * B + base, bpw)],
                            nidxT_v.at[n])

        def remap1d(j, _):
            pidx_v[pl.ds(j * LANES, LANES)] = _remap16(
                pidx_v[pl.ds(j * LANES, LANES)])
            return 0
        lax.fori_loop(0, bpw // LANES, remap1d, 0)
        for n in range(NNEG):
            def bodyn(j, _, n=n):
                nidxT_v[n, pl.ds(j * LANES, LANES)] = _remap16(
                    nidxT_v[n, pl.ds(j * LANES, LANES)])
                return 0
            lax.fori_loop(0, bpw // LANES, bodyn, 0)

        zero = jnp.zeros((LANES,), jnp.float32)

        def chunk_body(c, _):
            cb = c * CB

            def zbody(i, _):
                n_sum[i, pl.ds(0, LANES)] = zero
                n_sum[i, pl.ds(LANES, LANES)] = zero
                return 0
            lax.fori_loop(0, CB, zbody, 0)

            copies = [pltpu.async_copy(
                V_hbm.at[pidx_v.at[pl.ds(cb, CB)]], p_rows, sem)]
            for n in range(NNEG):
                copies.append(pltpu.async_copy(
                    V_hbm.at[nidxT_v.at[n, pl.ds(cb, CB)]],
                    n_sum, sem, add=True))
            for cp in copies:
                cp.wait()

            pltpu.sync_copy(p_rows, prow_out.at[pl.ds(base + cb, CB)])
            pltpu.sync_copy(n_sum, nsum_out.at[pl.ds(base + cb, CB)])
            return 0

        lax.fori_loop(0, nch, chunk_body, 0)

    return sc_vrows


def _sc_dots_builder(B):
    bpw = B // NW
    nch = bpw // CB

    @functools.partial(
        pl.kernel,
        out_type=(jax.ShapeDtypeStruct((B,), jnp.float32),
                  jax.ShapeDtypeStruct((B,), jnp.float32)),
        mesh=_sc_mesh(),
        compiler_params=_SC_PARAMS,
        scratch_types=[
            pltpu.VMEM((bpw,), jnp.int32),
            pltpu.VMEM((CB, DIM), jnp.float32),
            pltpu.VMEM((CB, DIM), jnp.float32),
            pltpu.VMEM((CB, DIM), jnp.float32),
            pltpu.VMEM((bpw,), jnp.float32),
            pltpu.VMEM((bpw,), jnp.float32),
            pltpu.SemaphoreType.DMA,
        ],
    )
    def sc_dots(U_hbm, uidx_hbm, prow_hbm, nsum_hbm,
                pos_out, neg_out,
                uidx_v, u_rows, p_rows, n_sum, pos_sv, neg_sv, sem):
        wid = lax.axis_index("s") * NC + lax.axis_index("c")
        base = wid * bpw

        pltpu.sync_copy(uidx_hbm.at[pl.ds(base, bpw)], uidx_v)

        def remap1d(j, _):
            uidx_v[pl.ds(j * LANES, LANES)] = _remap16(
                uidx_v[pl.ds(j * LANES, LANES)])
            return 0
        lax.fori_loop(0, bpw // LANES, remap1d, 0)

        lane = lax.iota(jnp.int32, LANES)
        zero = jnp.zeros((LANES,), jnp.float32)

        def chunk_body(c, _):
            cb = c * CB
            copies = [
                pltpu.async_copy(U_hbm.at[uidx_v.at[pl.ds(cb, CB)]],
                                 u_rows, sem),
                pltpu.async_copy(prow_hbm.at[pl.ds(base + cb, CB)],
                                 p_rows, sem),
                pltpu.async_copy(nsum_hbm.at[pl.ds(base + cb, CB)],
                                 n_sum, sem),
            ]
            for cp in copies:
                cp.wait()

            for g in range(CB // LANES):
                b_ids = lane + (g * LANES)

                def d_body(d, carry):
                    pos_vec, neg_vec = carry
                    d_ids = jnp.full((LANES,), d, jnp.int32)
                    u_d = plsc.load_gather(u_rows, [b_ids, d_ids])
                    p_d = plsc.load_gather(p_rows, [b_ids, d_ids])
                    ns_d = plsc.load_gather(n_sum, [b_ids, d_ids])
                    return (pos_vec + u_d * p_d, neg_vec + u_d * ns_d)

                pos_vec, neg_vec = lax.fori_loop(0, DIM, d_body, (zero, zero))
                pos_sv[pl.ds(cb + g * LANES, LANES)] = pos_vec
                neg_sv[pl.ds(cb + g * LANES, LANES)] = neg_vec
            return 0

        lax.fori_loop(0, nch, chunk_body, 0)

        pltpu.sync_copy(pos_sv, pos_out.at[pl.ds(base, bpw)])
        pltpu.sync_copy(neg_sv, neg_out.at[pl.ds(base, bpw)])

    return sc_dots


def _tc_loss_body(pos_ref, neg_ref, bs_ref, out_ref):
    pos = pos_ref[...]
    neg = -neg_ref[...]
    lp = jnp.minimum(pos, 0.0) - jnp.log1p(jnp.exp(-jnp.abs(pos)))
    ln = jnp.minimum(neg, 0.0) - jnp.log1p(jnp.exp(-jnp.abs(neg)))
    out_ref[0, 0] = -jnp.sum(lp + ln) / bs_ref[0]


def kernel(U, V, u_pos, v_pos, v_neg, batch_size):
    B = u_pos.shape[0]
    uidx = u_pos.astype(jnp.int32)
    pidx = v_pos.astype(jnp.int32)
    nidxT = jnp.transpose(v_neg.astype(jnp.int32)).reshape(-1)

    Vp = _pack_table(V)
    prows, nsums = _sc_vrows_builder(B)(Vp, pidx, nidxT)
    Up = _pack_table(U)
    pos_s, neg_s = _sc_dots_builder(B)(Up, uidx, prows, nsums)

    rows = B // 128
    bs = jnp.asarray(batch_size, jnp.float32).reshape(1)
    loss = pl.pallas_call(
        _tc_loss_body,
        out_shape=jax.ShapeDtypeStruct((1, 1), jnp.float32),
        in_specs=[
            pl.BlockSpec(memory_space=pltpu.VMEM),
            pl.BlockSpec(memory_space=pltpu.VMEM),
            pl.BlockSpec(memory_space=pltpu.SMEM),
        ],
        out_specs=pl.BlockSpec(memory_space=pltpu.SMEM),
    )(pos_s.reshape(rows, 128), neg_s.reshape(rows, 128), bs)
    return loss.reshape(())
